# baseline (speedup 1.0000x reference)
import jax
import jax.numpy as jnp
from jax import lax
from jax.experimental import pallas as pl
from jax.experimental.pallas import tpu as pltpu

_NMS_THRESH = 0.6
_SCORE_THRESH = 0.1
_DETECTIONS = 100
_NEG = -1e9

_N = 20000
_P = 20480
_ROWS = _P // 128
_CHUNKS = _ROWS // 8


def _nms_body(msc, mx1, my1, mx2, my2, ox1, oy1, ox2, oy2, osc, ooff,
              outf_ref, outk_ref, m_scr, a2_scr):
    m_scr[...] = msc[...]
    a2_scr[...] = (mx2[...] - mx1[...]) * (my2[...] - my1[...])

    lin8 = (lax.broadcasted_iota(jnp.int32, (8, 128), 0) * 128
            + lax.broadcasted_iota(jnp.int32, (8, 128), 1))
    lane = lax.broadcasted_iota(jnp.int32, (1, 128), 1)
    lane8 = lax.broadcasted_iota(jnp.int32, (8, 128), 1)
    negf = jnp.float32(_NEG)

    vmax = jnp.full((8, 128), -2e9, jnp.float32)
    vidx = jnp.zeros((8, 128), jnp.int32)
    zf8 = jnp.zeros((8, 128), jnp.float32)
    vx1, vy1, vx2, vy2, vsc, voff = zf8, zf8, zf8, zf8, zf8, zf8
    for i in range(_CHUNKS):
        sl = pl.ds(8 * i, 8)
        mc = m_scr[sl]
        upd = mc > vmax
        vmax = jnp.maximum(vmax, mc)
        vidx = jnp.where(upd, lin8 + 1024 * i, vidx)
        vx1 = jnp.where(upd, ox1[sl], vx1)
        vy1 = jnp.where(upd, oy1[sl], vy1)
        vx2 = jnp.where(upd, ox2[sl], vx2)
        vy2 = jnp.where(upd, oy2[sl], vy2)
        vsc = jnp.where(upd, osc[sl], vsc)
        voff = jnp.where(upd, ooff[sl], voff)

    zf = jnp.zeros((1, 128), jnp.float32)
    zi = jnp.zeros((1, 128), jnp.int32)

    def argmax_butterfly(av, ai):
        for axis, steps in ((1, 7), (0, 3)):
            for j in range(steps):
                bv = pltpu.roll(av, 1 << j, axis)
                bi = pltpu.roll(ai, 1 << j, axis)
                take = jnp.logical_or(
                    bv > av, jnp.logical_and(bv == av, bi < ai))
                av = jnp.where(take, bv, av)
                ai = jnp.where(take, bi, ai)
        return av, ai

    def allsum(x):
        for axis, steps in ((1, 7), (0, 3)):
            for j in range(steps):
                x = x + pltpu.roll(x, 1 << j, axis)
        return x

    def step(t, carry):
        (vmax, vidx, vx1, vy1, vx2, vy2, vsc, voff,
         keep_acc, x1_acc, y1_acc, x2_acc, y2_acc, sc_acc) = carry
        _, idx = argmax_butterfly(vmax, vidx)
        em = vidx == idx

        bx1 = allsum(jnp.where(em, vx1, 0.0))
        by1 = allsum(jnp.where(em, vy1, 0.0))
        bx2 = allsum(jnp.where(em, vx2, 0.0))
        by2 = allsum(jnp.where(em, vy2, 0.0))
        bsc = allsum(jnp.where(em, vsc, 0.0))
        boff = allsum(jnp.where(em, voff, 0.0))
        sx1 = bx1 + boff
        sy1 = by1 + boff
        sx2 = bx2 + boff
        sy2 = by2 + boff
        a1 = (sx2 - sx1) * (sy2 - sy1)

        nvmax = jnp.full((8, 128), -2e9, jnp.float32)
        nvidx = jnp.zeros((8, 128), jnp.int32)
        nvx1, nvy1, nvx2, nvy2, nvsc, nvoff = zf8, zf8, zf8, zf8, zf8, zf8
        for i in range(_CHUNKS):
            sl = pl.ds(8 * i, 8)
            x1 = jnp.maximum(sx1, mx1[sl])
            y1 = jnp.maximum(sy1, my1[sl])
            x2 = jnp.minimum(sx2, mx2[sl])
            y2 = jnp.minimum(sy2, my2[sl])
            inter = jnp.maximum(x2 - x1, 0.0) * jnp.maximum(y2 - y1, 0.0)
            iou = inter / (a1 + a2_scr[sl] - inter + 1e-9)
            newm = jnp.where(iou > _NMS_THRESH, negf, m_scr[sl])
            m_scr[sl] = newm
            upd = newm > nvmax
            nvmax = jnp.maximum(nvmax, newm)
            nvidx = jnp.where(upd, lin8 + 1024 * i, nvidx)
            nvx1 = jnp.where(upd, ox1[sl], nvx1)
            nvy1 = jnp.where(upd, oy1[sl], nvy1)
            nvx2 = jnp.where(upd, ox2[sl], nvx2)
            nvy2 = jnp.where(upd, oy2[sl], nvy2)
            nvsc = jnp.where(upd, osc[sl], nvsc)
            nvoff = jnp.where(upd, ooff[sl], nvoff)

        sel_t = lane8 == t
        return (nvmax, nvidx, nvx1, nvy1, nvx2, nvy2, nvsc, nvoff,
                jnp.where(sel_t, idx, keep_acc),
                jnp.where(sel_t, bx1, x1_acc),
                jnp.where(sel_t, by1, y1_acc),
                jnp.where(sel_t, bx2, x2_acc),
                jnp.where(sel_t, by2, y2_acc),
                jnp.where(sel_t, bsc, sc_acc))

    zi8 = jnp.zeros((8, 128), jnp.int32)
    carry0 = (vmax, vidx, vx1, vy1, vx2, vy2, vsc, voff,
              zi8, zf8, zf8, zf8, zf8, zf8)
    res = lax.fori_loop(0, _DETECTIONS, step, carry0)
    keep_acc, x1_acc, y1_acc, x2_acc, y2_acc, sc_acc = res[8:]

    outk_ref[...] = keep_acc
    outf_ref[...] = jnp.concatenate(
        [x1_acc[0:1], y1_acc[0:1], x2_acc[0:1], y2_acc[0:1], sc_acc[0:1],
         zf, zf, zf], axis=0)


@jax.jit
def kernel(boxes, scores, labels):
    off = labels.astype(boxes.dtype) * 4000.0
    msc = jnp.where(scores > _SCORE_THRESH, scores, _NEG)

    pad = _P - _N

    def pad1(x, val):
        return jnp.concatenate([x, jnp.full((pad,), val, x.dtype)])

    mscp = pad1(msc, _NEG).reshape(_ROWS, 128)
    offp = pad1(off, 0.0)
    b = [pad1(boxes[:, i], 0.0) for i in range(4)]
    mx = [(bi + offp).reshape(_ROWS, 128) for bi in b]
    o = [bi.reshape(_ROWS, 128) for bi in b]
    oscp = pad1(scores, 0.0).reshape(_ROWS, 128)
    ooffp = offp.reshape(_ROWS, 128)

    outf, outk = pl.pallas_call(
        _nms_body,
        out_shape=[jax.ShapeDtypeStruct((8, 128), jnp.float32),
                   jax.ShapeDtypeStruct((8, 128), jnp.int32)],
        scratch_shapes=[pltpu.VMEM((_ROWS, 128), jnp.float32),
                        pltpu.VMEM((_ROWS, 128), jnp.float32)],
    )(mscp, *mx, *o, oscp, ooffp)

    keep = outk[0, :_DETECTIONS]
    out = jnp.stack([outf[i, :_DETECTIONS] for i in range(5)], axis=1)
    return out, keep


# speedup vs baseline: 1.9237x; 1.9237x over previous
import jax
import jax.numpy as jnp
from jax import lax
from jax.experimental import pallas as pl
from jax.experimental.pallas import tpu as pltpu

_NMS_THRESH = 0.6
_SCORE_THRESH = 0.1
_DETECTIONS = 100
_NEG = -1e9

_N = 20000
_P = 20480
_ROWS = _P // 128
_CHUNKS = _ROWS // 8


def _nms_body(msc, mx1, my1, mx2, my2, ox1, oy1, ox2, oy2, osc, ooff,
              outf_ref, outk_ref, m_scr, a2_scr):
    m_scr[...] = msc[...]
    a2_scr[...] = (mx2[...] - mx1[...]) * (my2[...] - my1[...])

    lin8 = (lax.broadcasted_iota(jnp.int32, (8, 128), 0) * 128
            + lax.broadcasted_iota(jnp.int32, (8, 128), 1))
    lane = lax.broadcasted_iota(jnp.int32, (1, 128), 1)
    lane8 = lax.broadcasted_iota(jnp.int32, (8, 128), 1)
    negf = jnp.float32(_NEG)

    vmax = jnp.full((8, 128), -2e9, jnp.float32)
    vidx = jnp.zeros((8, 128), jnp.int32)
    zf8 = jnp.zeros((8, 128), jnp.float32)
    vx1, vy1, vx2, vy2, vsc, voff = zf8, zf8, zf8, zf8, zf8, zf8
    for i in range(_CHUNKS):
        sl = pl.ds(8 * i, 8)
        mc = m_scr[sl]
        upd = mc > vmax
        vmax = jnp.maximum(vmax, mc)
        vidx = jnp.where(upd, lin8 + 1024 * i, vidx)
        vx1 = jnp.where(upd, ox1[sl], vx1)
        vy1 = jnp.where(upd, oy1[sl], vy1)
        vx2 = jnp.where(upd, ox2[sl], vx2)
        vy2 = jnp.where(upd, oy2[sl], vy2)
        vsc = jnp.where(upd, osc[sl], vsc)
        voff = jnp.where(upd, ooff[sl], voff)

    zf = jnp.zeros((1, 128), jnp.float32)
    zi = jnp.zeros((1, 128), jnp.int32)

    def splat(x):
        return jnp.broadcast_to(
            jnp.max(x, axis=(0, 1), keepdims=True), (8, 128))

    def splat_min(x):
        return jnp.broadcast_to(
            jnp.min(x, axis=(0, 1), keepdims=True), (8, 128))

    def splat_sum(x):
        return jnp.broadcast_to(
            jnp.sum(x, axis=(0, 1), keepdims=True), (8, 128))

    def step(t, carry):
        (vmax, vidx, vx1, vy1, vx2, vy2, vsc, voff,
         keep_acc, x1_acc, y1_acc, x2_acc, y2_acc, sc_acc) = carry
        big = jnp.int32(2**30)
        mv = splat(vmax)
        idx = splat_min(jnp.where(vmax == mv, vidx, big))
        em = vidx == idx

        bx1 = splat_sum(jnp.where(em, vx1, 0.0))
        by1 = splat_sum(jnp.where(em, vy1, 0.0))
        bx2 = splat_sum(jnp.where(em, vx2, 0.0))
        by2 = splat_sum(jnp.where(em, vy2, 0.0))
        bsc = splat_sum(jnp.where(em, vsc, 0.0))
        boff = splat_sum(jnp.where(em, voff, 0.0))
        sx1 = bx1 + boff
        sy1 = by1 + boff
        sx2 = bx2 + boff
        sy2 = by2 + boff
        a1 = (sx2 - sx1) * (sy2 - sy1)

        nvmax = jnp.full((8, 128), -2e9, jnp.float32)
        nvidx = jnp.zeros((8, 128), jnp.int32)
        nvx1, nvy1, nvx2, nvy2, nvsc, nvoff = zf8, zf8, zf8, zf8, zf8, zf8
        for i in range(_CHUNKS):
            sl = pl.ds(8 * i, 8)
            x1 = jnp.maximum(sx1, mx1[sl])
            y1 = jnp.maximum(sy1, my1[sl])
            x2 = jnp.minimum(sx2, mx2[sl])
            y2 = jnp.minimum(sy2, my2[sl])
            inter = jnp.maximum(x2 - x1, 0.0) * jnp.maximum(y2 - y1, 0.0)
            iou = inter / (a1 + a2_scr[sl] - inter + 1e-9)
            newm = jnp.where(iou > _NMS_THRESH, negf, m_scr[sl])
            m_scr[sl] = newm
            upd = newm > nvmax
            nvmax = jnp.maximum(nvmax, newm)
            nvidx = jnp.where(upd, lin8 + 1024 * i, nvidx)
            nvx1 = jnp.where(upd, ox1[sl], nvx1)
            nvy1 = jnp.where(upd, oy1[sl], nvy1)
            nvx2 = jnp.where(upd, ox2[sl], nvx2)
            nvy2 = jnp.where(upd, oy2[sl], nvy2)
            nvsc = jnp.where(upd, osc[sl], nvsc)
            nvoff = jnp.where(upd, ooff[sl], nvoff)

        sel_t = lane8 == t
        return (nvmax, nvidx, nvx1, nvy1, nvx2, nvy2, nvsc, nvoff,
                jnp.where(sel_t, idx, keep_acc),
                jnp.where(sel_t, bx1, x1_acc),
                jnp.where(sel_t, by1, y1_acc),
                jnp.where(sel_t, bx2, x2_acc),
                jnp.where(sel_t, by2, y2_acc),
                jnp.where(sel_t, bsc, sc_acc))

    zi8 = jnp.zeros((8, 128), jnp.int32)
    carry0 = (vmax, vidx, vx1, vy1, vx2, vy2, vsc, voff,
              zi8, zf8, zf8, zf8, zf8, zf8)
    res = lax.fori_loop(0, _DETECTIONS, step, carry0)
    keep_acc, x1_acc, y1_acc, x2_acc, y2_acc, sc_acc = res[8:]

    outk_ref[...] = keep_acc
    outf_ref[...] = jnp.concatenate(
        [x1_acc[0:1], y1_acc[0:1], x2_acc[0:1], y2_acc[0:1], sc_acc[0:1],
         zf, zf, zf], axis=0)


@jax.jit
def kernel(boxes, scores, labels):
    off = labels.astype(boxes.dtype) * 4000.0
    msc = jnp.where(scores > _SCORE_THRESH, scores, _NEG)

    pad = _P - _N

    def pad1(x, val):
        return jnp.concatenate([x, jnp.full((pad,), val, x.dtype)])

    mscp = pad1(msc, _NEG).reshape(_ROWS, 128)
    offp = pad1(off, 0.0)
    b = [pad1(boxes[:, i], 0.0) for i in range(4)]
    mx = [(bi + offp).reshape(_ROWS, 128) for bi in b]
    o = [bi.reshape(_ROWS, 128) for bi in b]
    oscp = pad1(scores, 0.0).reshape(_ROWS, 128)
    ooffp = offp.reshape(_ROWS, 128)

    outf, outk = pl.pallas_call(
        _nms_body,
        out_shape=[jax.ShapeDtypeStruct((8, 128), jnp.float32),
                   jax.ShapeDtypeStruct((8, 128), jnp.int32)],
        scratch_shapes=[pltpu.VMEM((_ROWS, 128), jnp.float32),
                        pltpu.VMEM((_ROWS, 128), jnp.float32)],
    )(mscp, *mx, *o, oscp, ooffp)

    keep = outk[0, :_DETECTIONS]
    out = jnp.stack([outf[i, :_DETECTIONS] for i in range(5)], axis=1)
    return out, keep
